# baseline (speedup 1.0000x reference)
import jax
import jax.numpy as jnp
from jax import lax
from jax.experimental import pallas as pl
from jax.experimental.pallas import tpu as pltpu
from jax.experimental.pallas import tpu_sc as plsc

_M, _D, _B = 100000, 128, 16384
_CW = 4
_NCG = _D // _CW
_NCH = 4
_RC = _M // _NCH
_S = 1024
_PW = _S * _CW
_NP = _B // _S
_NV = _PW // 16
_Q = _RC // 4


def _body(inp_hbm, idx_hbm, src_hbm, out_hbm,
          acc, ib0, ib1, sb0, sb1, sem0, sem1):
  cid = lax.axis_index("c")
  sid = lax.axis_index("s")
  cg = cid * 16 + sid
  hc0 = cid * 64
  lanes = lax.iota(jnp.int32, 16)
  col = lanes & 3

  def start(piece, ib, sb, sem):
    pltpu.async_copy(idx_hbm.at[cg, pl.ds(piece * _PW, _PW)], ib, sem)
    pltpu.async_copy(src_hbm.at[cg, pl.ds(piece * _PW, _PW)], sb, sem)

  def wait(piece, ib, sb, sem):
    pltpu.make_async_copy(
        idx_hbm.at[cg, pl.ds(piece * _PW, _PW)], ib, sem).wait()
    pltpu.make_async_copy(
        src_hbm.at[cg, pl.ds(piece * _PW, _PW)], sb, sem).wait()

  def consume(r0, ib, sb):
    @plsc.parallel_loop(0, _NV, unroll=16)
    def vec(v):
      iv = ib[pl.ds(v * 16, 16)]
      sv = sb[pl.ds(v * 16, 16)]
      loc = iv - r0
      msk = (loc >= 0) & (loc < _RC)
      addr = (loc << 2) | col
      plsc.addupdate_scatter(acc, [addr], sv, mask=msk)

  def chunk(ch, carry):
    r0 = ch * _RC

    pltpu.sync_copy(inp_hbm.at[cg, pl.ds(r0 * _CW, _RC * _CW)], acc)

    start(0, ib0, sb0, sem0)

    def pair(j, c2):
      pa = 2 * j
      pb = pa + 1
      start(pb, ib1, sb1, sem1)
      wait(pa, ib0, sb0, sem0)
      consume(r0, ib0, sb0)

      @pl.when(j + 1 < _NP // 2)
      def _():
        start(pa + 2, ib0, sb0, sem0)

      wait(pb, ib1, sb1, sem1)
      consume(r0, ib1, sb1)
      return c2

    lax.fori_loop(0, _NP // 2, pair, 0)

    pltpu.sync_copy(acc, out_hbm.at[cg, pl.ds(r0 * _CW, _RC * _CW)])
    return carry

  lax.fori_loop(0, _NCH, chunk, 0)


@jax.jit
def _scatter_add(inp, idx, src):
  idx_b = idx.reshape(_B, _NCG, _CW).transpose(1, 0, 2).reshape(_NCG, _B * _CW)
  src_b = src.reshape(_B, _NCG, _CW).transpose(1, 0, 2).reshape(_NCG, _B * _CW)
  inp_b = inp.reshape(_M, _NCG, _CW).transpose(1, 0, 2).reshape(_NCG, _M * _CW)
  mesh = plsc.VectorSubcoreMesh(core_axis_name="c", subcore_axis_name="s")
  run = pl.kernel(
      _body,
      out_type=jax.ShapeDtypeStruct((_NCG, _M * _CW), jnp.float32),
      mesh=mesh,
      compiler_params=pltpu.CompilerParams(use_tc_tiling_on_sc=False,
                                           needs_layout_passes=False),
      scratch_types=[
          pltpu.VMEM((_RC * _CW,), jnp.float32),
          pltpu.VMEM((_PW,), jnp.int32),
          pltpu.VMEM((_PW,), jnp.int32),
          pltpu.VMEM((_PW,), jnp.float32),
          pltpu.VMEM((_PW,), jnp.float32),
          pltpu.SemaphoreType.DMA,
          pltpu.SemaphoreType.DMA,
      ],
  )
  out_b = run(inp_b, idx_b, src_b)
  return out_b.reshape(_NCG, _M, _CW).transpose(1, 0, 2).reshape(_M, _D)


def kernel(input, dim, index, src):
  idx = (index + dim).astype(jnp.int32)
  return _scatter_add(input, idx, src)


# speedup vs baseline: 1.8466x; 1.8466x over previous
import functools

import jax
import jax.numpy as jnp
from jax import lax
from jax.experimental import pallas as pl
from jax.experimental.pallas import tpu as pltpu
from jax.experimental.pallas import tpu_sc as plsc

_M, _D, _B = 100000, 128, 16384
_CW = 16
_NCG = _D // _CW
_NRP = 4
_RPR = _M // _NRP
_NCH = 4
_R = _RPR // _NCH
_S = 256
_NP = _B // _S


def _body(inp_hbm, scan_hbm, out_hbm,
          acc, ib0, ib1, sb0, sb1, sem0, sem1):
  cid = lax.axis_index("c")
  sid = lax.axis_index("s")
  wid = sid * 2 + cid
  cg = wid % _NCG
  rp = wid // _NCG
  c0 = cg * _CW
  lanes = lax.iota(jnp.int32, _CW)

  def start(piece, ib, sb, sem):
    pltpu.async_copy(scan_hbm.at[cg, pl.ds(piece * _S, _S), :], ib, sem)
    pltpu.async_copy(scan_hbm.at[cg, pl.ds(_B + piece * _S, _S), :], sb, sem)

  def wait(piece, ib, sb, sem):
    pltpu.make_async_copy(
        scan_hbm.at[cg, pl.ds(piece * _S, _S), :], ib, sem).wait()
    pltpu.make_async_copy(
        scan_hbm.at[cg, pl.ds(_B + piece * _S, _S), :], sb, sem).wait()

  def consume(r0, ib, sb):
    @plsc.parallel_loop(0, _S, unroll=16)
    def row(r):
      iv = ib[r]
      sv = plsc.bitcast(sb[r], jnp.float32)
      loc = iv - r0
      msk = plsc.bitcast(loc, jnp.uint32) < jnp.uint32(_R)
      plsc.addupdate_scatter(acc, [loc, lanes], sv, mask=msk)

  def chunk(ch, carry):
    r0 = rp * _RPR + ch * _R
    pltpu.sync_copy(inp_hbm.at[pl.ds(r0, _R), pl.ds(c0, _CW)], acc)
    start(0, ib0, sb0, sem0)

    def pair(j, c2):
      pa = 2 * j
      pb = pa + 1
      start(pb, ib1, sb1, sem1)
      wait(pa, ib0, sb0, sem0)
      consume(r0, ib0, sb0)

      @pl.when(j + 1 < _NP // 2)
      def _():
        start(pa + 2, ib0, sb0, sem0)

      wait(pb, ib1, sb1, sem1)
      consume(r0, ib1, sb1)
      return c2

    lax.fori_loop(0, _NP // 2, pair, 0)
    pltpu.sync_copy(acc, out_hbm.at[pl.ds(r0, _R), pl.ds(c0, _CW)])
    return carry

  lax.fori_loop(0, _NCH, chunk, 0)


@jax.jit
def _scatter_add(inp, idx, src):
  src_i = jax.lax.bitcast_convert_type(src, jnp.int32)
  scan = jnp.concatenate([idx, src_i], axis=0)
  scan_b = scan.reshape(2 * _B, _NCG, _CW).transpose(1, 0, 2)
  mesh = plsc.VectorSubcoreMesh(core_axis_name="c", subcore_axis_name="s")
  run = pl.kernel(
      _body,
      out_type=jax.ShapeDtypeStruct((_M, _D), jnp.float32),
      mesh=mesh,
      compiler_params=pltpu.CompilerParams(use_tc_tiling_on_sc=False,
                           needs_layout_passes=False),
      scratch_types=[
          pltpu.VMEM((_R, _CW), jnp.float32),
          pltpu.VMEM((_S, _CW), jnp.int32),
          pltpu.VMEM((_S, _CW), jnp.int32),
          pltpu.VMEM((_S, _CW), jnp.int32),
          pltpu.VMEM((_S, _CW), jnp.int32),
          pltpu.SemaphoreType.DMA,
          pltpu.SemaphoreType.DMA,
      ],
  )
  return run(inp, scan_b)


def kernel(input, dim, index, src):
  idx = (index + dim).astype(jnp.int32)
  return _scatter_add(input, idx, src)


# speedup vs baseline: 2.4757x; 1.3407x over previous
import jax
import jax.numpy as jnp
from jax import lax
from jax.experimental import pallas as pl
from jax.experimental.pallas import tpu as pltpu
from jax.experimental.pallas import tpu_sc as plsc

_M, _D, _B = 100000, 128, 16384
_CW = 16
_NCG = _D // _CW
_NRP = 4
_RPR = _M // _NRP
_NCH = 4
_R = _RPR // _NCH
_S = 256
_NP = _B // _S
_NB = 3


def _body(inp_hbm, scan_hbm, out_hbm, acc, b0, b1, b2, s0, s1, s2):
  bufs = (b0, b1, b2)
  sems = (s0, s1, s2)
  cid = lax.axis_index("c")
  sid = lax.axis_index("s")
  wid = sid * 2 + cid
  cg = wid % _NCG
  rp = wid // _NCG
  c0 = cg * _CW
  lanes = lax.iota(jnp.int32, 16)

  def start(piece, t):
    pltpu.async_copy(
        scan_hbm.at[pl.ds(piece * _S, _S), pl.ds(cg * 2 * _CW, 2 * _CW)],
        bufs[t], sems[t])

  def wait(piece, t):
    pltpu.make_async_copy(
        scan_hbm.at[pl.ds(piece * _S, _S), pl.ds(cg * 2 * _CW, 2 * _CW)],
        bufs[t], sems[t]).wait()

  def consume(r0, t):
    buf = bufs[t]

    @plsc.parallel_loop(0, _S, unroll=16)
    def row(r):
      iv = buf[r, pl.ds(0, _CW)]
      sv = plsc.bitcast(buf[r, pl.ds(_CW, _CW)], jnp.float32)
      loc = iv - r0
      msk = plsc.bitcast(loc, jnp.uint32) < jnp.uint32(_R)
      plsc.addupdate_scatter(acc, [loc, lanes], sv, mask=msk)

  def chunk(ch, carry):
    r0 = rp * _RPR + ch * _R
    pltpu.sync_copy(inp_hbm.at[pl.ds(r0, _R), pl.ds(c0, _CW)], acc)

    for t in range(_NB):
      start(t, t)

    def triple(j, c2):
      p = 3 * j
      for t in range(_NB):
        wait(p + t, t)
        consume(r0, t)

        @pl.when(p + t + _NB < _NP)
        def _():
          start(p + t + _NB, t)
      return c2

    lax.fori_loop(0, (_NP - 1) // _NB, triple, 0)
    wait(_NP - 1, (_NP - 1) % _NB)
    consume(r0, (_NP - 1) % _NB)

    pltpu.sync_copy(acc, out_hbm.at[pl.ds(r0, _R), pl.ds(c0, _CW)])
    return carry

  lax.fori_loop(0, _NCH, chunk, 0)


@jax.jit
def _scatter_add(inp, idx, src):
  src_i = jax.lax.bitcast_convert_type(src, jnp.int32)
  ixs = jnp.stack(
      [idx.reshape(_B, _NCG, _CW), src_i.reshape(_B, _NCG, _CW)],
      axis=2).reshape(_B, 2 * _D)
  mesh = plsc.VectorSubcoreMesh(core_axis_name="c", subcore_axis_name="s")
  run = pl.kernel(
      _body,
      out_type=jax.ShapeDtypeStruct((_M, _D), jnp.float32),
      mesh=mesh,
      compiler_params=pltpu.CompilerParams(use_tc_tiling_on_sc=False,
                                           needs_layout_passes=False),
      scratch_types=[
          pltpu.VMEM((_R, _CW), jnp.float32),
          pltpu.VMEM((_S, 2 * _CW), jnp.int32),
          pltpu.VMEM((_S, 2 * _CW), jnp.int32),
          pltpu.VMEM((_S, 2 * _CW), jnp.int32),
          pltpu.SemaphoreType.DMA,
          pltpu.SemaphoreType.DMA,
          pltpu.SemaphoreType.DMA,
      ],
  )
  return run(inp, ixs)


def kernel(input, dim, index, src):
  idx = (index + dim).astype(jnp.int32)
  return _scatter_add(input, idx, src)


# speedup vs baseline: 3.0460x; 1.2304x over previous
import jax
import jax.numpy as jnp
from jax import lax
from jax.experimental import pallas as pl
from jax.experimental.pallas import tpu as pltpu
from jax.experimental.pallas import tpu_sc as plsc

_M, _D, _B = 100000, 128, 16384
_CW = 16
_NCG = _D // _CW
_NRP = 4
_RPR = _M // _NRP
_NCH = 4
_R = _RPR // _NCH
_S = 256
_NP = _B // _S
_NB = 3


def _body(inp_hbm, idx_hbm, src_hbm, out_hbm,
          acc, b0, b1, b2, c0b, c1b, c2b, s0, s1, s2):
  ibufs = (b0, b1, b2)
  sbufs = (c0b, c1b, c2b)
  sems = (s0, s1, s2)
  cid = lax.axis_index("c")
  sid = lax.axis_index("s")
  wid = sid * 2 + cid
  cg = wid % _NCG
  rp = wid // _NCG
  c0 = cg * _CW
  lanes = lax.iota(jnp.int32, 16)

  def start(piece, t):
    pltpu.async_copy(
        idx_hbm.at[pl.ds(piece * _S, _S), pl.ds(c0, _CW)], ibufs[t], sems[t])
    pltpu.async_copy(
        src_hbm.at[pl.ds(piece * _S, _S), pl.ds(c0, _CW)], sbufs[t], sems[t])

  def wait(piece, t):
    pltpu.make_async_copy(
        idx_hbm.at[pl.ds(piece * _S, _S), pl.ds(c0, _CW)],
        ibufs[t], sems[t]).wait()
    pltpu.make_async_copy(
        src_hbm.at[pl.ds(piece * _S, _S), pl.ds(c0, _CW)],
        sbufs[t], sems[t]).wait()

  def consume(r0, t):
    ib = ibufs[t]
    sb = sbufs[t]

    @plsc.parallel_loop(0, _S, unroll=16)
    def row(r):
      iv = ib[r]
      sv = sb[r]
      loc = iv - r0
      msk = plsc.bitcast(loc, jnp.uint32) < jnp.uint32(_R)
      plsc.addupdate_scatter(acc, [loc, lanes], sv, mask=msk)

  def chunk(ch, carry):
    r0 = rp * _RPR + ch * _R
    pltpu.sync_copy(inp_hbm.at[pl.ds(r0, _R), pl.ds(c0, _CW)], acc)

    for t in range(_NB):
      start(t, t)

    def triple(j, c2):
      p = 3 * j
      for t in range(_NB):
        wait(p + t, t)
        consume(r0, t)

        @pl.when(p + t + _NB < _NP)
        def _():
          start(p + t + _NB, t)
      return c2

    lax.fori_loop(0, (_NP - 1) // _NB, triple, 0)
    wait(_NP - 1, (_NP - 1) % _NB)
    consume(r0, (_NP - 1) % _NB)

    pltpu.sync_copy(acc, out_hbm.at[pl.ds(r0, _R), pl.ds(c0, _CW)])
    return carry

  lax.fori_loop(0, _NCH, chunk, 0)


@jax.jit
def _scatter_add(inp, idx, src):
  mesh = plsc.VectorSubcoreMesh(core_axis_name="c", subcore_axis_name="s")
  run = pl.kernel(
      _body,
      out_type=jax.ShapeDtypeStruct((_M, _D), jnp.float32),
      mesh=mesh,
      compiler_params=pltpu.CompilerParams(use_tc_tiling_on_sc=False,
                                           needs_layout_passes=False),
      scratch_types=[
          pltpu.VMEM((_R, _CW), jnp.float32),
          pltpu.VMEM((_S, _CW), jnp.int32),
          pltpu.VMEM((_S, _CW), jnp.int32),
          pltpu.VMEM((_S, _CW), jnp.int32),
          pltpu.VMEM((_S, _CW), jnp.float32),
          pltpu.VMEM((_S, _CW), jnp.float32),
          pltpu.VMEM((_S, _CW), jnp.float32),
          pltpu.SemaphoreType.DMA,
          pltpu.SemaphoreType.DMA,
          pltpu.SemaphoreType.DMA,
      ],
  )
  return run(inp, idx, src)


def kernel(input, dim, index, src):
  idx = (index + dim).astype(jnp.int32)
  return _scatter_add(input, idx, src)


# speedup vs baseline: 3.0992x; 1.0175x over previous
import jax
import jax.numpy as jnp
from jax import lax
from jax.experimental import pallas as pl
from jax.experimental.pallas import tpu as pltpu
from jax.experimental.pallas import tpu_sc as plsc

_M, _D, _B = 100000, 128, 16384
_CW = 16
_NCG = _D // _CW
_NRP = 4
_RPR = _M // _NRP
_NCH = 4
_R = _RPR // _NCH
_S = 256
_NP = _B // _S
_NB = 3


def _body(inp_hbm, idx_hbm, src_hbm, out_hbm,
          acc, b0, b1, b2, c0b, c1b, c2b, s0, s1, s2):
  ibufs = (b0, b1, b2)
  sbufs = (c0b, c1b, c2b)
  sems = (s0, s1, s2)
  cid = lax.axis_index("c")
  sid = lax.axis_index("s")
  wid = sid * 2 + cid
  cg = wid % _NCG
  rp = wid // _NCG
  c0 = cg * _CW
  lanes = lax.iota(jnp.int32, 16)

  def start(piece, t):
    pltpu.async_copy(
        idx_hbm.at[pl.ds(piece * _S, _S), pl.ds(c0, _CW)], ibufs[t], sems[t])
    pltpu.async_copy(
        src_hbm.at[pl.ds(piece * _S, _S), pl.ds(c0, _CW)], sbufs[t], sems[t])

  def wait(piece, t):
    pltpu.make_async_copy(
        idx_hbm.at[pl.ds(piece * _S, _S), pl.ds(c0, _CW)],
        ibufs[t], sems[t]).wait()
    pltpu.make_async_copy(
        src_hbm.at[pl.ds(piece * _S, _S), pl.ds(c0, _CW)],
        sbufs[t], sems[t]).wait()

  def consume(r0, t):
    ib = ibufs[t]
    sb = sbufs[t]

    @plsc.parallel_loop(0, _S, unroll=16)
    def row(r):
      iv = ib[r]
      sv = sb[r]
      loc = iv - r0
      msk = plsc.bitcast(loc, jnp.uint32) < jnp.uint32(_R)
      plsc.addupdate_scatter(acc, [loc, lanes], sv, mask=msk)

  def chunk(ch, carry):
    r0 = rp * _RPR + ch * _R
    pltpu.sync_copy(inp_hbm.at[pl.ds(r0, _R), pl.ds(c0, _CW)], acc)

    def triple(j, c2):
      p = 3 * j
      for t in range(_NB):
        wait(p + t, t)
        consume(r0, t)

        @pl.when(p + t + _NB < _NP)
        def _():
          start(p + t + _NB, t)
      return c2

    lax.fori_loop(0, (_NP - 1) // _NB, triple, 0)
    wait(_NP - 1, (_NP - 1) % _NB)
    consume(r0, (_NP - 1) % _NB)

    for t in range(_NB):
      start(t, t)

    pltpu.sync_copy(acc, out_hbm.at[pl.ds(r0, _R), pl.ds(c0, _CW)])
    return carry

  for t in range(_NB):
    start(t, t)
  lax.fori_loop(0, _NCH, chunk, 0)
  for t in range(_NB):
    wait(t, t)


@jax.jit
def _scatter_add(inp, idx, src):
  mesh = plsc.VectorSubcoreMesh(core_axis_name="c", subcore_axis_name="s")
  run = pl.kernel(
      _body,
      out_type=jax.ShapeDtypeStruct((_M, _D), jnp.float32),
      mesh=mesh,
      compiler_params=pltpu.CompilerParams(use_tc_tiling_on_sc=False,
                                           needs_layout_passes=False),
      scratch_types=[
          pltpu.VMEM((_R, _CW), jnp.float32),
          pltpu.VMEM((_S, _CW), jnp.int32),
          pltpu.VMEM((_S, _CW), jnp.int32),
          pltpu.VMEM((_S, _CW), jnp.int32),
          pltpu.VMEM((_S, _CW), jnp.float32),
          pltpu.VMEM((_S, _CW), jnp.float32),
          pltpu.VMEM((_S, _CW), jnp.float32),
          pltpu.SemaphoreType.DMA,
          pltpu.SemaphoreType.DMA,
          pltpu.SemaphoreType.DMA,
      ],
  )
  return run(inp, idx, src)


def kernel(input, dim, index, src):
  idx = (index + dim).astype(jnp.int32)
  return _scatter_add(input, idx, src)


# speedup vs baseline: 3.1852x; 1.0277x over previous
import jax
import jax.numpy as jnp
from jax import lax
from jax.experimental import pallas as pl
from jax.experimental.pallas import tpu as pltpu
from jax.experimental.pallas import tpu_sc as plsc

_M, _D, _B = 100000, 128, 16384
_CW = 16
_NCG = _D // _CW
_NRP = 4
_RPR = _M // _NRP
_NCH = 4
_R = _RPR // _NCH
_S = 256
_NP = _B // _S
_NB = 3


def _body(inp_hbm, idx_hbm, src_hbm, out_hbm,
          acc, b0, b1, b2, c0b, c1b, c2b, s0, s1, s2, semw0, semw1):
  ibufs = (b0, b1, b2)
  sbufs = (c0b, c1b, c2b)
  sems = (s0, s1, s2)
  cid = lax.axis_index("c")
  sid = lax.axis_index("s")
  wid = sid * 2 + cid
  cg = wid % _NCG
  rp = wid // _NCG
  c0 = cg * _CW
  lanes = lax.iota(jnp.int32, 16)

  def start(piece, t):
    pltpu.async_copy(
        idx_hbm.at[pl.ds(piece * _S, _S), pl.ds(c0, _CW)], ibufs[t], sems[t])
    pltpu.async_copy(
        src_hbm.at[pl.ds(piece * _S, _S), pl.ds(c0, _CW)], sbufs[t], sems[t])

  def wait(piece, t):
    pltpu.make_async_copy(
        idx_hbm.at[pl.ds(piece * _S, _S), pl.ds(c0, _CW)],
        ibufs[t], sems[t]).wait()
    pltpu.make_async_copy(
        src_hbm.at[pl.ds(piece * _S, _S), pl.ds(c0, _CW)],
        sbufs[t], sems[t]).wait()

  def consume(r0, t):
    ib = ibufs[t]
    sb = sbufs[t]

    @plsc.parallel_loop(0, _S, unroll=16)
    def row(r):
      iv = ib[r]
      sv = sb[r]
      loc = iv - r0
      msk = plsc.bitcast(loc, jnp.uint32) < jnp.uint32(_R)
      plsc.addupdate_scatter(acc, [loc, lanes], sv, mask=msk)

  def chunk(ch, carry):
    r0 = rp * _RPR + ch * _R

    def triple(j, c2):
      p = 3 * j
      for t in range(_NB):
        wait(p + t, t)
        consume(r0, t)

        @pl.when(p + t + _NB < _NP)
        def _():
          start(p + t + _NB, t)
      return c2

    lax.fori_loop(0, (_NP - 1) // _NB, triple, 0)
    wait(_NP - 1, (_NP - 1) % _NB)
    consume(r0, (_NP - 1) % _NB)

    for t in range(_NB):
      start(t, t)

    h = _R // 2
    r1 = r0 + _R
    wa = pltpu.make_async_copy(
        acc.at[pl.ds(0, h), :], out_hbm.at[pl.ds(r0, h), pl.ds(c0, _CW)],
        semw0)
    wb = pltpu.make_async_copy(
        acc.at[pl.ds(h, h), :], out_hbm.at[pl.ds(r0 + h, h), pl.ds(c0, _CW)],
        semw1)
    wa.start()
    wb.start()
    wa.wait()

    @pl.when(ch + 1 < _NCH)
    def _():
      pltpu.sync_copy(inp_hbm.at[pl.ds(r1, h), pl.ds(c0, _CW)],
                      acc.at[pl.ds(0, h), :])
    wb.wait()

    @pl.when(ch + 1 < _NCH)
    def _():
      pltpu.sync_copy(inp_hbm.at[pl.ds(r1 + h, h), pl.ds(c0, _CW)],
                      acc.at[pl.ds(h, h), :])
    return carry

  for t in range(_NB):
    start(t, t)
  pltpu.sync_copy(inp_hbm.at[pl.ds(rp * _RPR, _R), pl.ds(c0, _CW)], acc)
  lax.fori_loop(0, _NCH, chunk, 0)
  for t in range(_NB):
    wait(t, t)


@jax.jit
def _scatter_add(inp, idx, src):
  mesh = plsc.VectorSubcoreMesh(core_axis_name="c", subcore_axis_name="s")
  run = pl.kernel(
      _body,
      out_type=jax.ShapeDtypeStruct((_M, _D), jnp.float32),
      mesh=mesh,
      compiler_params=pltpu.CompilerParams(use_tc_tiling_on_sc=False,
                                           needs_layout_passes=False),
      scratch_types=[
          pltpu.VMEM((_R, _CW), jnp.float32),
          pltpu.VMEM((_S, _CW), jnp.int32),
          pltpu.VMEM((_S, _CW), jnp.int32),
          pltpu.VMEM((_S, _CW), jnp.int32),
          pltpu.VMEM((_S, _CW), jnp.float32),
          pltpu.VMEM((_S, _CW), jnp.float32),
          pltpu.VMEM((_S, _CW), jnp.float32),
          pltpu.SemaphoreType.DMA,
          pltpu.SemaphoreType.DMA,
          pltpu.SemaphoreType.DMA,
          pltpu.SemaphoreType.DMA,
          pltpu.SemaphoreType.DMA,
      ],
  )
  return run(inp, idx, src)


def kernel(input, dim, index, src):
  idx = (index + dim).astype(jnp.int32)
  return _scatter_add(input, idx, src)
